# baseline (speedup 1.0000x reference)
import jax
import jax.numpy as jnp
from jax.experimental import pallas as pl
from jax.experimental.pallas import tpu as pltpu

_B = 1024
_C, _H, _W = 4, 64, 64
_COLS = _C * _H * _W
_CB = 32
_NCH = _B // _CB


def _fwd_kernel(t_sref, sac_sref, somac_sref, x_ref, n_ref, xt_ref, tg_ref):
    c = pl.program_id(0)
    rows = jax.lax.broadcasted_iota(jnp.int32, (_CB, 1), 0)
    sa = jnp.zeros((_CB, 1), jnp.float32)
    so = jnp.zeros((_CB, 1), jnp.float32)
    for i in range(_CB):
        ti = t_sref[c * _CB + i]
        sa = jnp.where(rows == i, sac_sref[ti], sa)
        so = jnp.where(rows == i, somac_sref[ti], so)
    x = x_ref[...]
    n = n_ref[...]
    xt_ref[...] = sa * x + so * n
    tg_ref[...] = sa * n - so * x


def kernel(x0, t, noise, sqrt_alphas_cumprod, sqrt_one_minus_alphas_cumprod):
    t32 = t.astype(jnp.int32)
    x2 = x0.reshape(_B, _COLS)
    n2 = noise.reshape(_B, _COLS)

    grid_spec = pltpu.PrefetchScalarGridSpec(
        num_scalar_prefetch=3,
        grid=(_NCH,),
        in_specs=[
            pl.BlockSpec((_CB, _COLS), lambda c, *_: (c, 0)),
            pl.BlockSpec((_CB, _COLS), lambda c, *_: (c, 0)),
        ],
        out_specs=[
            pl.BlockSpec((_CB, _COLS), lambda c, *_: (c, 0)),
            pl.BlockSpec((_CB, _COLS), lambda c, *_: (c, 0)),
        ],
    )
    xt, tgt = pl.pallas_call(
        _fwd_kernel,
        grid_spec=grid_spec,
        compiler_params=pltpu.CompilerParams(
            dimension_semantics=("parallel",),
        ),
        out_shape=[
            jax.ShapeDtypeStruct((_B, _COLS), jnp.float32),
            jax.ShapeDtypeStruct((_B, _COLS), jnp.float32),
        ],
    )(t32, sqrt_alphas_cumprod, sqrt_one_minus_alphas_cumprod, x2, n2)
    return xt.reshape(_B, _C, _H, _W), tgt.reshape(_B, _C, _H, _W)
